# baseline (speedup 1.0000x reference)
import jax
import jax.numpy as jnp
from jax.experimental import pallas as pl
from jax.experimental.pallas import tpu as pltpu

DEPTH_RATIO = 0.5
MM_DT = jnp.bfloat16
F_TILES = 2
TT = 512


def _silu(z):
    return z * jax.nn.sigmoid(z)


def _moe_kernel(tau_ref, probe_W_ref, probe_b_ref, xT_ref,
                wup_ref, wgate_ref, wdown_ref, out_ref, c_ref):
    e = pl.program_id(0)
    f = pl.program_id(1)
    step = e * pl.num_programs(1) + f
    T = out_ref.shape[1]

    @pl.when(step == 0)
    def _router():
        logits = jax.lax.dot_general(
            probe_W_ref[...], xT_ref[...], (((1,), (0,)), ((), ())),
            preferred_element_type=jnp.float32)
        logits = logits + probe_b_ref[...]
        tau = tau_ref[0]
        c_ref[...] = jnp.where(logits > tau, jax.nn.sigmoid(logits),
                               jnp.zeros_like(logits))

    wup = wup_ref[0].astype(MM_DT)
    wgate = wgate_ref[0].astype(MM_DT)
    wdown = wdown_ref[0].astype(MM_DT)

    for t in range(T // TT):
        ts = pl.ds(t * TT, TT)
        xa = xT_ref[:, ts]
        up = jax.lax.dot_general(
            wup, xa, (((1,), (0,)), ((), ())),
            preferred_element_type=jnp.float32)
        gate = jax.lax.dot_general(
            wgate, xa, (((1,), (0,)), ((), ())),
            preferred_element_type=jnp.float32)
        c_row = c_ref[pl.ds(e, 1), ts]
        h = (up * _silu(gate) * c_row).astype(MM_DT)
        dn = jax.lax.dot_general(
            wdown, h, (((1,), (0,)), ((), ())),
            preferred_element_type=jnp.float32)

        @pl.when(step == 0)
        def _():
            out_ref[:, ts] = dn

        @pl.when(step != 0)
        def _():
            out_ref[:, ts] += dn


def kernel(x, probe_W, probe_b, W_up, W_gate, W_down, tau_base, gamma, w_depth):
    T, D = x.shape
    E, D_FF, _ = W_up.shape
    F_t = D_FF // F_TILES

    z = w_depth * DEPTH_RATIO
    tau = (tau_base + gamma * (z * jax.nn.sigmoid(z))).astype(jnp.float32)
    tau = tau.reshape(1)

    xT = x.T.astype(MM_DT)
    pw = probe_W.astype(MM_DT)
    pb = probe_b.reshape(E, 1).astype(jnp.float32)

    outT = pl.pallas_call(
        _moe_kernel,
        grid=(E, F_TILES),
        in_specs=[
            pl.BlockSpec(memory_space=pltpu.SMEM),
            pl.BlockSpec((E, D), lambda e, f: (0, 0)),
            pl.BlockSpec((E, 1), lambda e, f: (0, 0)),
            pl.BlockSpec((D, T), lambda e, f: (0, 0)),
            pl.BlockSpec((1, F_t, D), lambda e, f: (e, f, 0)),
            pl.BlockSpec((1, F_t, D), lambda e, f: (e, f, 0)),
            pl.BlockSpec((1, D, F_t), lambda e, f: (e, 0, f)),
        ],
        out_specs=pl.BlockSpec((D, T), lambda e, f: (0, 0)),
        out_shape=jax.ShapeDtypeStruct((D, T), jnp.float32),
        scratch_shapes=[pltpu.VMEM((E, T), jnp.float32)],
        compiler_params=pltpu.CompilerParams(
            dimension_semantics=("arbitrary", "arbitrary"),
            vmem_limit_bytes=64 * 1024 * 1024,
        ),
    )(tau, pw, pb, xT, W_up, W_gate, W_down)
    return outT.T


# speedup vs baseline: 1.0702x; 1.0702x over previous
import jax
import jax.numpy as jnp
from jax.experimental import pallas as pl
from jax.experimental.pallas import tpu as pltpu

DEPTH_RATIO = 0.5
MM_DT = jnp.bfloat16
F_TILES = 2
TT = 512


def _silu(z):
    return z * jax.nn.sigmoid(z)


def _moe_kernel(tau_ref, probe_W_ref, probe_b_ref, xT_ref,
                wup_ref, wgate_ref, wdown_ref, out_ref, c_ref):
    e = pl.program_id(0)
    f = pl.program_id(1)
    step = e * pl.num_programs(1) + f
    T = out_ref.shape[0]

    @pl.when(step == 0)
    def _router():
        logits = jax.lax.dot_general(
            probe_W_ref[...], xT_ref[...], (((1,), (0,)), ((), ())),
            preferred_element_type=jnp.float32)
        logits = logits + probe_b_ref[...]
        tau = tau_ref[0]
        c_ref[...] = jnp.where(logits > tau, jax.nn.sigmoid(logits),
                               jnp.zeros_like(logits))

    wup = wup_ref[0].astype(MM_DT)
    wgate = wgate_ref[0].astype(MM_DT)
    wdown = wdown_ref[0].astype(MM_DT)

    for t in range(T // TT):
        ts = pl.ds(t * TT, TT)
        xa = xT_ref[:, ts]
        up = jax.lax.dot_general(
            wup, xa, (((1,), (0,)), ((), ())),
            preferred_element_type=jnp.float32)
        gate = jax.lax.dot_general(
            wgate, xa, (((1,), (0,)), ((), ())),
            preferred_element_type=jnp.float32)
        c_row = c_ref[pl.ds(e, 1), ts]
        h = (up * _silu(gate) * c_row).astype(MM_DT)
        dn = jax.lax.dot_general(
            h, wdown, (((0,), (1,)), ((), ())),
            preferred_element_type=jnp.float32)

        @pl.when(step == 0)
        def _():
            out_ref[ts, :] = dn

        @pl.when(step != 0)
        def _():
            out_ref[ts, :] += dn


def kernel(x, probe_W, probe_b, W_up, W_gate, W_down, tau_base, gamma, w_depth):
    T, D = x.shape
    E, D_FF, _ = W_up.shape
    F_t = D_FF // F_TILES

    z = w_depth * DEPTH_RATIO
    tau = (tau_base + gamma * (z * jax.nn.sigmoid(z))).astype(jnp.float32)
    tau = tau.reshape(1)

    xT = x.T.astype(MM_DT)
    pw = probe_W.astype(MM_DT)
    pb = probe_b.reshape(E, 1).astype(jnp.float32)

    outT = pl.pallas_call(
        _moe_kernel,
        grid=(E, F_TILES),
        in_specs=[
            pl.BlockSpec(memory_space=pltpu.SMEM),
            pl.BlockSpec((E, D), lambda e, f: (0, 0)),
            pl.BlockSpec((E, 1), lambda e, f: (0, 0)),
            pl.BlockSpec((D, T), lambda e, f: (0, 0)),
            pl.BlockSpec((1, F_t, D), lambda e, f: (e, f, 0)),
            pl.BlockSpec((1, F_t, D), lambda e, f: (e, f, 0)),
            pl.BlockSpec((1, D, F_t), lambda e, f: (e, 0, f)),
        ],
        out_specs=pl.BlockSpec((T, D), lambda e, f: (0, 0)),
        out_shape=jax.ShapeDtypeStruct((T, D), jnp.float32),
        scratch_shapes=[pltpu.VMEM((E, T), jnp.float32)],
        compiler_params=pltpu.CompilerParams(
            dimension_semantics=("arbitrary", "arbitrary"),
            vmem_limit_bytes=64 * 1024 * 1024,
        ),
    )(tau, pw, pb, xT, W_up, W_gate, W_down)
    return outT
